# baseline (speedup 1.0000x reference)
import functools

import jax
import jax.numpy as jnp
from jax import lax
from jax.experimental import pallas as pl
from jax.experimental.pallas import tpu as pltpu
from jax.experimental.pallas import tpu_sc as plsc

B = 4096
S = 200
D = 64
H = 32
NW = 32
BPW = B // NW
TPW = BPW * S
C0 = 128
C1 = S - C0


def _pool_body(table_hbm, tok_hbm, out_hbm, idx_v, buf0, buf1, acc_v, sem0, sem1):
    wid = lax.axis_index("s") * 2 + lax.axis_index("c")
    tok_off = pl.multiple_of(wid * TPW, 8)
    pltpu.sync_copy(tok_hbm.at[pl.ds(tok_off, TPW)], idx_v)

    def start(r, buf, sem):
        off = pl.multiple_of(r * S, 8)
        pltpu.async_copy(table_hbm.at[idx_v.at[pl.ds(off, C0)]],
                         buf.at[pl.ds(0, C0)], sem)
        off2 = pl.multiple_of(r * S + C0, 8)
        pltpu.async_copy(table_hbm.at[idx_v.at[pl.ds(off2, C1)]],
                         buf.at[pl.ds(C0, C1)], sem)

    def wait(buf, sem):
        pltpu.make_async_copy(table_hbm.at[pl.ds(0, S)], buf, sem).wait()

    def accum(buf, r):
        def body(k, a):
            return tuple(a[c] + buf[k, pl.ds(c * 16, 16)] for c in range(4))
        z = jnp.zeros((16,), jnp.float32)
        a = lax.fori_loop(0, S, body, (z, z, z, z), unroll=4)
        for c in range(4):
            acc_v[r, pl.ds(c * 16, 16)] = a[c]

    start(0, buf0, sem0)

    @pl.loop(0, BPW, step=2)
    def _(r):
        start(r + 1, buf1, sem1)
        wait(buf0, sem0)
        accum(buf0, r)

        @pl.when(r + 2 < BPW)
        def _():
            start(r + 2, buf0, sem0)

        wait(buf1, sem1)
        accum(buf1, r + 1)

    row_off = pl.multiple_of(wid * BPW, 8)
    pltpu.sync_copy(acc_v, out_hbm.at[pl.ds(row_off, BPW)])


def _pool(table, tok_flat):
    mesh = plsc.VectorSubcoreMesh(core_axis_name="c", subcore_axis_name="s")
    k = pl.kernel(
        _pool_body,
        out_type=jax.ShapeDtypeStruct((B, D), jnp.float32),
        mesh=mesh,
        compiler_params=pltpu.CompilerParams(use_tc_tiling_on_sc=False),
        scratch_types=[
            pltpu.VMEM((TPW,), jnp.int32),
            pltpu.VMEM((S, D), jnp.float32),
            pltpu.VMEM((S, D), jnp.float32),
            pltpu.VMEM((BPW, D), jnp.float32),
            pltpu.SemaphoreType.DMA,
            pltpu.SemaphoreType.DMA,
        ],
    )
    return k(table, tok_flat)


def _mlp_body(x_ref, w1t_ref, b1_ref, w2_ref, b2_ref, o_ref):
    x = x_ref[...]
    h = jnp.dot(x, w1t_ref[...], preferred_element_type=jnp.float32,
                precision=lax.Precision.HIGHEST)
    h = jnp.maximum(h + b1_ref[...], 0.0)
    o_ref[...] = jnp.sum(h * w2_ref[...], axis=1, keepdims=True) + b2_ref[...]


def _mlp(pooled_sum, W1, b1, W2, b2):
    w1t = W1.T * (1.0 / S)
    out = pl.pallas_call(
        _mlp_body,
        out_shape=jax.ShapeDtypeStruct((B, 1), jnp.float32),
    )(pooled_sum, w1t, b1.reshape(1, H), W2, b2.reshape(1, 1))
    return out[:, 0]


def kernel(tokens, table, W1, b1, W2, b2):
    tok_flat = tokens.reshape(-1).astype(jnp.int32)
    pooled_sum = _pool(table, tok_flat)
    return _mlp(pooled_sum, W1, b1, W2, b2)
